# baseline (speedup 1.0000x reference)
import functools
import math

import jax
import jax.numpy as jnp
from jax import lax
from jax.experimental import pallas as pl
from jax.experimental.pallas import tpu as pltpu
from jax.experimental.pallas import tpu_sc as plsc

D_MODEL = 64
SCALE = math.sqrt(D_MODEL)

_NC = 2
_NS = 16
_NW = _NC * _NS

_B = 4096 * 200
_B_W = _B // _NW
_CH = 128
_NSTEP = _B_W // _CH


def _body(x_hbm, table_hbm, out_hbm, idx_v, rows_v, gsem):
    wid = lax.axis_index("s") * _NC + lax.axis_index("c")
    base = wid * _B_W

    pltpu.sync_copy(x_hbm.at[wid], idx_v)

    def step(g, carry):
        pltpu.async_copy(table_hbm.at[idx_v.at[g]], rows_v, gsem).wait()

        def scale_row(r, c2):
            for c in range(D_MODEL // 16):
                sl = pl.ds(c * 16, 16)
                rows_v[r, sl] = rows_v[r, sl] * SCALE
            return c2

        lax.fori_loop(0, _CH, scale_row, 0, unroll=4)

        pltpu.sync_copy(rows_v, out_hbm.at[pl.ds(base + g * _CH, _CH)])
        return carry

    lax.fori_loop(0, _NSTEP, step, 0)


@functools.partial(
    pl.kernel,
    out_type=jax.ShapeDtypeStruct((_B, D_MODEL), jnp.float32),
    mesh=plsc.VectorSubcoreMesh(core_axis_name="c", subcore_axis_name="s"),
    compiler_params=pltpu.CompilerParams(use_tc_tiling_on_sc=False),
    scratch_types=[
        pltpu.VMEM((_NSTEP, _CH), jnp.int32),
        pltpu.VMEM((_CH, D_MODEL), jnp.float32),
        pltpu.SemaphoreType.DMA,
    ],
)
def _emb_lookup(x_hbm, table_hbm, out_hbm, idx_v, rows_v, gsem):
    _body(x_hbm, table_hbm, out_hbm, idx_v, rows_v, gsem)


@jax.jit
def kernel(x, table):
    xg = x.reshape(_NW, _NSTEP, _CH)
    out = _emb_lookup(xg, table)
    return out.reshape(x.shape[0], x.shape[1], D_MODEL)


# speedup vs baseline: 1.1621x; 1.1621x over previous
import functools
import math

import jax
import jax.numpy as jnp
from jax import lax
from jax.experimental import pallas as pl
from jax.experimental.pallas import tpu as pltpu
from jax.experimental.pallas import tpu_sc as plsc

D_MODEL = 64
SCALE = math.sqrt(D_MODEL)

_NC = 2
_NS = 16
_NW = _NC * _NS

_B = 4096 * 200
_B_W = _B // _NW
_CH = 128
_NSTEP = _B_W // _CH
_NBUF = 4
_PRE = 2
_NGRP = _NSTEP // _NBUF


def _body(x_hbm, table_hbm, out_hbm, idx_v, rows, gsems, osems, base):
    def gather(g, b):
        return pltpu.make_async_copy(
            table_hbm.at[idx_v.at[g]], rows[b], gsems[b])

    def scatter(g, b):
        return pltpu.make_async_copy(
            rows[b], out_hbm.at[pl.ds(base + g * _CH, _CH)], osems[b])

    pltpu.sync_copy(x_hbm, idx_v)

    for s in range(_PRE):
        gather(s, s % _NBUF).start()

    def group(grp, carry):
        for b in range(_NBUF):
            s = grp * _NBUF + b
            s_pre = s + _PRE
            b_pre = (b + _PRE) % _NBUF

            @pl.when(s_pre < _NSTEP)
            def _():
                @pl.when(s_pre >= _NBUF)
                def _():
                    scatter(s_pre - _NBUF, b_pre).wait()
                gather(s_pre, b_pre).start()

            gather(s, b).wait()

            def scale_row(r, c2):
                for c in range(D_MODEL // 16):
                    sl = pl.ds(c * 16, 16)
                    rows[b][r, sl] = rows[b][r, sl] * SCALE
                return c2

            lax.fori_loop(0, _CH, scale_row, 0, unroll=4)
            scatter(s, b).start()
        return carry

    lax.fori_loop(0, _NGRP, group, 0)

    for b in range(_NBUF):
        scatter(_NSTEP - _NBUF + b, b).wait()


@functools.partial(
    pl.kernel,
    out_type=jax.ShapeDtypeStruct((_B, D_MODEL), jnp.float32),
    mesh=plsc.VectorSubcoreMesh(core_axis_name="c", subcore_axis_name="s"),
    compiler_params=pltpu.CompilerParams(use_tc_tiling_on_sc=False),
    scratch_types=[
        pltpu.VMEM((_NSTEP, _CH), jnp.int32),
        [pltpu.VMEM((_CH, D_MODEL), jnp.float32) for _ in range(_NBUF)],
        [pltpu.SemaphoreType.DMA for _ in range(_NBUF)],
        [pltpu.SemaphoreType.DMA for _ in range(_NBUF)],
    ],
)
def _emb_lookup(x_hbm, table_hbm, out_hbm, idx_v, rows, gsems, osems):
    wid = lax.axis_index("s") * _NC + lax.axis_index("c")
    _body(x_hbm.at[wid], table_hbm, out_hbm, idx_v, rows,
          gsems, osems, wid * _B_W)


@jax.jit
def kernel(x, table):
    xg = x.reshape(_NW, _NSTEP, _CH)
    out = _emb_lookup(xg, table)
    return out.reshape(x.shape[0], x.shape[1], D_MODEL)


# speedup vs baseline: 1.1631x; 1.0009x over previous
import functools
import math

import jax
import jax.numpy as jnp
from jax import lax
from jax.experimental import pallas as pl
from jax.experimental.pallas import tpu as pltpu
from jax.experimental.pallas import tpu_sc as plsc

D_MODEL = 64
SCALE = math.sqrt(D_MODEL)

_NC = 2
_NS = 16
_NW = _NC * _NS

_B = 4096 * 200
_B_W = _B // _NW
_CH = 128
_NSTEP = _B_W // _CH
_NBUF = 4
_PRE = 2
_NGRP = _NSTEP // _NBUF


def _body(x_hbm, table_hbm, out_hbm, idx_v, rows, gsems, osems, base):
    def gather(g, b):
        return pltpu.make_async_copy(
            table_hbm.at[idx_v.at[g]], rows[b], gsems[b])

    def scatter(g, b):
        return pltpu.make_async_copy(
            rows[b], out_hbm.at[pl.ds(base + g * _CH, _CH)], osems[b])

    pltpu.sync_copy(x_hbm, idx_v)

    for s in range(_PRE):
        gather(s, s % _NBUF).start()

    def group(grp, carry):
        for b in range(_NBUF):
            s = grp * _NBUF + b
            s_pre = s + _PRE
            b_pre = (b + _PRE) % _NBUF

            @pl.when(s_pre < _NSTEP)
            def _():
                @pl.when(s_pre >= _NBUF)
                def _():
                    scatter(s_pre - _NBUF, b_pre).wait()
                gather(s_pre, b_pre).start()

            gather(s, b).wait()

            buf = rows[b]

            @plsc.parallel_loop(0, _CH, step=1, unroll=8)
            def _scale(r):
                for c in range(D_MODEL // 16):
                    sl = pl.ds(c * 16, 16)
                    buf[r, sl] = buf[r, sl] * SCALE
            scatter(s, b).start()
        return carry

    lax.fori_loop(0, _NGRP, group, 0)

    for b in range(_NBUF):
        scatter(_NSTEP - _NBUF + b, b).wait()


@functools.partial(
    pl.kernel,
    out_type=jax.ShapeDtypeStruct((_B, D_MODEL), jnp.float32),
    mesh=plsc.VectorSubcoreMesh(core_axis_name="c", subcore_axis_name="s"),
    compiler_params=pltpu.CompilerParams(use_tc_tiling_on_sc=False),
    scratch_types=[
        pltpu.VMEM((_NSTEP, _CH), jnp.int32),
        [pltpu.VMEM((_CH, D_MODEL), jnp.float32) for _ in range(_NBUF)],
        [pltpu.SemaphoreType.DMA for _ in range(_NBUF)],
        [pltpu.SemaphoreType.DMA for _ in range(_NBUF)],
    ],
)
def _emb_lookup(x_hbm, table_hbm, out_hbm, idx_v, rows, gsems, osems):
    wid = lax.axis_index("s") * _NC + lax.axis_index("c")
    _body(x_hbm.at[wid], table_hbm, out_hbm, idx_v, rows,
          gsems, osems, wid * _B_W)


@jax.jit
def kernel(x, table):
    xg = x.reshape(_NW, _NSTEP, _CH)
    out = _emb_lookup(xg, table)
    return out.reshape(x.shape[0], x.shape[1], D_MODEL)
